# baseline (speedup 1.0000x reference)
import functools

import jax
import jax.numpy as jnp
from jax import lax
from jax.experimental import pallas as pl
from jax.experimental.pallas import tpu as pltpu
from jax.experimental.pallas import tpu_sc as plsc

N_NODES = 10000
N_PAD = 10240
N_EDGES = 160000
NUM_GRAPHS = 64
NC, NS = 2, 16
NW = NC * NS
CHUNK = 125
NCH = 40
CROWS = N_EDGES // CHUNK
ZROWS = N_PAD // NS
ZCH = 128

_mesh = lambda: plsc.VectorSubcoreMesh(core_axis_name="c", subcore_axis_name="s")


def _sc_degree(dst2d):
    @functools.partial(
        pl.kernel,
        out_type=jax.ShapeDtypeStruct((NC, N_PAD), jnp.float32),
        mesh=_mesh(),
        compiler_params=pltpu.CompilerParams(use_tc_tiling_on_sc=False),
        scratch_types=(
            pltpu.VMEM((NCH, CHUNK), jnp.int32),
            pltpu.VMEM((128,), jnp.float32),
            pltpu.VMEM((ZROWS,), jnp.float32),
            pltpu.SemaphoreType.DMA,
            pltpu.VMEM_SHARED((N_PAD,), jnp.float32),
        ),
    )
    def deg_kernel(dst_hbm, out_hbm, didx, ones_v, zbuf, sem, deg_sh):
        cid = lax.axis_index("c")
        sid = lax.axis_index("s")
        wid = sid * NC + cid
        zeros16 = jnp.zeros((16,), jnp.float32)
        ones16 = jnp.ones((16,), jnp.float32)

        def zb(i, c):
            zbuf[pl.ds(i * 16, 16)] = zeros16
            return c

        lax.fori_loop(0, ZROWS // 16, zb, 0)

        def ob(i, c):
            ones_v[pl.ds(i * 16, 16)] = ones16
            return c

        lax.fori_loop(0, 128 // 16, ob, 0)
        pltpu.sync_copy(zbuf, deg_sh.at[pl.ds(sid * ZROWS, ZROWS)])
        plsc.subcore_barrier()
        pltpu.sync_copy(dst_hbm.at[pl.ds(wid * NCH, NCH)], didx)

        def ch(i, c):
            pltpu.sync_copy(ones_v.at[pl.ds(0, CHUNK)],
                            deg_sh.at[didx.at[i]], add=True)
            return c

        lax.fori_loop(0, NCH, ch, 0)
        plsc.subcore_barrier()
        pltpu.sync_copy(deg_sh.at[pl.ds(sid * ZROWS, ZROWS)],
                        out_hbm.at[cid, pl.ds(sid * ZROWS, ZROWS)])

    return deg_kernel(dst2d)


def _sc_aggregate(h, src2d, dst2d):
    D = h.shape[1]
    dt = h.dtype
    lanes = 32 if dt == jnp.bfloat16 else 16

    @functools.partial(
        pl.kernel,
        out_type=jax.ShapeDtypeStruct((NC, N_PAD, D), dt),
        mesh=_mesh(),
        compiler_params=pltpu.CompilerParams(use_tc_tiling_on_sc=False),
        scratch_types=(
            pltpu.VMEM((NCH, CHUNK), jnp.int32),
            pltpu.VMEM((NCH, CHUNK), jnp.int32),
            pltpu.VMEM((CHUNK, D), dt),
            pltpu.VMEM((CHUNK, D), dt),
            pltpu.SemaphoreType.DMA,
            pltpu.SemaphoreType.DMA,
            pltpu.VMEM_SHARED((N_PAD, D), dt),
        ),
    )
    def agg_kernel(h_hbm, src_hbm, dst_hbm, out_hbm,
                   sidx, didx, rows_a, rows_b, sem_a, sem_b, acc_sh):
        cid = lax.axis_index("c")
        sid = lax.axis_index("s")
        wid = sid * NC + cid
        zvec = jnp.zeros((lanes,), dt)

        def zb(i, c):
            for j in range(D // lanes):
                rows_a[i, pl.ds(j * lanes, lanes)] = zvec
            return c

        lax.fori_loop(0, CHUNK, zb, 0)
        for k in range(ZROWS // CHUNK):
            pltpu.sync_copy(
                rows_a, acc_sh.at[pl.ds(sid * ZROWS + k * CHUNK, CHUNK)])
        pltpu.sync_copy(
            rows_a.at[pl.ds(0, ZROWS - (ZROWS // CHUNK) * CHUNK)],
            acc_sh.at[pl.ds(sid * ZROWS + (ZROWS // CHUNK) * CHUNK,
                            ZROWS - (ZROWS // CHUNK) * CHUNK)])
        plsc.subcore_barrier()
        r0 = wid * NCH
        pltpu.sync_copy(src_hbm.at[pl.ds(r0, NCH)], sidx)
        pltpu.sync_copy(dst_hbm.at[pl.ds(r0, NCH)], didx)

        def gather_a(j):
            return pltpu.make_async_copy(h_hbm.at[sidx.at[j]], rows_a, sem_a)

        def gather_b(j):
            return pltpu.make_async_copy(h_hbm.at[sidx.at[j]], rows_b, sem_b)

        gather_a(0).start()
        gather_b(1).start()
        npair = NCH // 2

        def body(i, c):
            gather_a(2 * i).wait()
            pltpu.sync_copy(rows_a, acc_sh.at[didx.at[2 * i]], add=True)

            @pl.when(i < npair - 1)
            def _():
                gather_a(2 * i + 2).start()

            gather_b(2 * i + 1).wait()
            pltpu.sync_copy(rows_b, acc_sh.at[didx.at[2 * i + 1]], add=True)

            @pl.when(i < npair - 1)
            def _():
                gather_b(2 * i + 3).start()

            return c

        lax.fori_loop(0, npair, body, 0)
        plsc.subcore_barrier()
        pltpu.sync_copy(acc_sh.at[pl.ds(sid * ZROWS, ZROWS)],
                        out_hbm.at[cid, pl.ds(sid * ZROWS, ZROWS)])

    return agg_kernel(h, src2d, dst2d)


_PREC = lax.Precision.DEFAULT
_RB = 2000


def _tc_dense1(x, W1, deg2d):
    K, F = x.shape[1], W1.shape[1]

    def body(x_ref, w_ref, deg_ref, ht_ref, dinv_ref):
        dinv = lax.rsqrt(deg_ref[...])
        h = jnp.dot(x_ref[...], w_ref[...],
                    preferred_element_type=jnp.float32, precision=_PREC)
        ht_ref[...] = (h * dinv).astype(jnp.bfloat16)
        dinv_ref[...] = dinv

    return pl.pallas_call(
        body,
        grid=(N_NODES // _RB,),
        in_specs=[
            pl.BlockSpec((_RB, K), lambda i: (i, 0)),
            pl.BlockSpec((K, F), lambda i: (0, 0)),
            pl.BlockSpec((_RB, 1), lambda i: (i, 0)),
        ],
        out_specs=(
            pl.BlockSpec((_RB, F), lambda i: (i, 0)),
            pl.BlockSpec((_RB, 1), lambda i: (i, 0)),
        ),
        out_shape=(jax.ShapeDtypeStruct((N_NODES, F), jnp.bfloat16),
                   jax.ShapeDtypeStruct((N_NODES, 1), jnp.float32)),
    )(x, W1, deg2d)


def _tc_dense2(S1p, ht1, dinv2d, b1, W2):
    K, F = W2.shape

    def body(sp_ref, ht_ref, dinv_ref, b_ref, w_ref, out_ref):
        s = (sp_ref[0].astype(jnp.float32) + sp_ref[1].astype(jnp.float32)
             + ht_ref[...].astype(jnp.float32))
        dinv = dinv_ref[...]
        a = jnp.maximum(s * dinv + b_ref[...], 0.0)
        h2 = jnp.dot(a, w_ref[...],
                     preferred_element_type=jnp.float32, precision=_PREC)
        out_ref[...] = (h2 * dinv).astype(jnp.bfloat16)

    return pl.pallas_call(
        body,
        grid=(N_NODES // _RB,),
        in_specs=[
            pl.BlockSpec((2, _RB, K), lambda i: (0, i, 0)),
            pl.BlockSpec((_RB, K), lambda i: (i, 0)),
            pl.BlockSpec((_RB, 1), lambda i: (i, 0)),
            pl.BlockSpec((1, K), lambda i: (0, 0)),
            pl.BlockSpec((K, F), lambda i: (0, 0)),
        ],
        out_specs=pl.BlockSpec((_RB, F), lambda i: (i, 0)),
        out_shape=jax.ShapeDtypeStruct((N_NODES, F), jnp.bfloat16),
    )(S1p, ht1, dinv2d, b1, W2)


def _tc_final(S2p, ht2, dinv2d, b2, batch_row, Wl, bl):
    def body(sp_ref, ht_ref, dinv_ref, b_ref, batch_ref, wl_ref, bl_ref,
             out_ref):
        s = (sp_ref[0, :N_NODES, :].astype(jnp.float32)
             + sp_ref[1, :N_NODES, :].astype(jnp.float32)
             + ht_ref[...].astype(jnp.float32))
        h = jnp.maximum(s * dinv_ref[...] + b_ref[...], 0.0)
        gids = lax.broadcasted_iota(jnp.int32, (NUM_GRAPHS, N_NODES), 0)
        ohT = (gids == batch_ref[...]).astype(jnp.float32)
        sums = jnp.dot(ohT, h, preferred_element_type=jnp.float32,
                       precision=_PREC)
        counts = jnp.sum(ohT, axis=1, keepdims=True)
        g = sums / jnp.maximum(counts, 1.0)
        out_ref[...] = jnp.dot(g, wl_ref[...],
                               preferred_element_type=jnp.float32,
                               precision=_PREC) + bl_ref[...]

    return pl.pallas_call(
        body,
        out_shape=jax.ShapeDtypeStruct((NUM_GRAPHS, Wl.shape[1]), jnp.float32),
    )(S2p, ht2, dinv2d, b2, batch_row, Wl, bl)


def kernel(x, edge_index, batch, W1, b1, W2, b2, Wl, bl):
    src2d = edge_index[0].astype(jnp.int32).reshape(CROWS, CHUNK)
    dst2d = edge_index[1].astype(jnp.int32).reshape(CROWS, CHUNK)
    batch_row = batch.astype(jnp.int32).reshape(1, N_NODES)
    degp = _sc_degree(dst2d)
    deg2d = (degp[0, :N_NODES] + degp[1, :N_NODES] + 1.0)[:, None]
    ht1, dinv2d = _tc_dense1(x, W1, deg2d)
    S1p = _sc_aggregate(ht1, src2d, dst2d)
    ht2 = _tc_dense2(S1p, ht1, dinv2d, b1.reshape(1, -1), W2)
    S2p = _sc_aggregate(ht2, src2d, dst2d)
    return _tc_final(S2p, ht2, dinv2d, b2.reshape(1, -1), batch_row,
                     Wl, bl.reshape(1, -1))


# speedup vs baseline: 1.0579x; 1.0579x over previous
import functools

import jax
import jax.numpy as jnp
from jax import lax
from jax.experimental import pallas as pl
from jax.experimental.pallas import tpu as pltpu
from jax.experimental.pallas import tpu_sc as plsc

N_NODES = 10000
N_PAD = 10240
N_EDGES = 160000
NUM_GRAPHS = 64
NC, NS = 2, 16
NW = NC * NS
CHUNK = 125
NCH = 40
CROWS = N_EDGES // CHUNK
ZROWS = N_PAD // NS
ZCH = 128

_mesh = lambda: plsc.VectorSubcoreMesh(core_axis_name="c", subcore_axis_name="s")


def _sc_degree(dst2d):
    @functools.partial(
        pl.kernel,
        out_type=jax.ShapeDtypeStruct((NC, N_PAD), jnp.float32),
        mesh=_mesh(),
        compiler_params=pltpu.CompilerParams(use_tc_tiling_on_sc=False),
        scratch_types=(
            pltpu.VMEM((NCH, CHUNK), jnp.int32),
            pltpu.VMEM((128,), jnp.float32),
            pltpu.VMEM((ZROWS,), jnp.float32),
            pltpu.SemaphoreType.DMA,
            pltpu.VMEM_SHARED((N_PAD,), jnp.float32),
        ),
    )
    def deg_kernel(dst_hbm, out_hbm, didx, ones_v, zbuf, sem, deg_sh):
        cid = lax.axis_index("c")
        sid = lax.axis_index("s")
        wid = sid * NC + cid
        zeros16 = jnp.zeros((16,), jnp.float32)
        ones16 = jnp.ones((16,), jnp.float32)

        def zb(i, c):
            zbuf[pl.ds(i * 16, 16)] = zeros16
            return c

        lax.fori_loop(0, ZROWS // 16, zb, 0)

        def ob(i, c):
            ones_v[pl.ds(i * 16, 16)] = ones16
            return c

        lax.fori_loop(0, 128 // 16, ob, 0)
        pltpu.sync_copy(zbuf, deg_sh.at[pl.ds(sid * ZROWS, ZROWS)])
        plsc.subcore_barrier()
        pltpu.sync_copy(dst_hbm.at[pl.ds(wid * NCH, NCH)], didx)

        def ch(i, c):
            pltpu.sync_copy(ones_v.at[pl.ds(0, CHUNK)],
                            deg_sh.at[didx.at[i]], add=True)
            return c

        lax.fori_loop(0, NCH, ch, 0)
        plsc.subcore_barrier()
        pltpu.sync_copy(deg_sh.at[pl.ds(sid * ZROWS, ZROWS)],
                        out_hbm.at[cid, pl.ds(sid * ZROWS, ZROWS)])

    return deg_kernel(dst2d)


def _sc_aggregate(h, src2d, dst2d):
    D = h.shape[1]
    dt = h.dtype
    lanes = 32 if dt == jnp.bfloat16 else 16

    @functools.partial(
        pl.kernel,
        out_type=jax.ShapeDtypeStruct((NC, N_PAD, D), dt),
        mesh=_mesh(),
        compiler_params=pltpu.CompilerParams(use_tc_tiling_on_sc=False),
        scratch_types=(
            pltpu.VMEM((NCH, CHUNK), jnp.int32),
            pltpu.VMEM((NCH, CHUNK), jnp.int32),
            pltpu.VMEM((CHUNK, D), dt),
            pltpu.VMEM((CHUNK, D), dt),
            pltpu.SemaphoreType.DMA,
            pltpu.SemaphoreType.DMA,
            pltpu.VMEM_SHARED((N_PAD, D), dt),
        ),
    )
    def agg_kernel(h_hbm, src_hbm, dst_hbm, out_hbm,
                   sidx, didx, rows_a, rows_b, sem_a, sem_b, acc_sh):
        cid = lax.axis_index("c")
        sid = lax.axis_index("s")
        wid = sid * NC + cid
        zvec = jnp.zeros((lanes,), dt)

        def zb(i, c):
            for j in range(D // lanes):
                rows_a[i, pl.ds(j * lanes, lanes)] = zvec
            return c

        lax.fori_loop(0, CHUNK, zb, 0)
        for k in range(ZROWS // CHUNK):
            pltpu.sync_copy(
                rows_a, acc_sh.at[pl.ds(sid * ZROWS + k * CHUNK, CHUNK)])
        pltpu.sync_copy(
            rows_a.at[pl.ds(0, ZROWS - (ZROWS // CHUNK) * CHUNK)],
            acc_sh.at[pl.ds(sid * ZROWS + (ZROWS // CHUNK) * CHUNK,
                            ZROWS - (ZROWS // CHUNK) * CHUNK)])
        plsc.subcore_barrier()
        r0 = wid * NCH
        pltpu.sync_copy(src_hbm.at[pl.ds(r0, NCH)], sidx)
        pltpu.sync_copy(dst_hbm.at[pl.ds(r0, NCH)], didx)

        def gather_a(j):
            return pltpu.make_async_copy(h_hbm.at[sidx.at[j]], rows_a, sem_a)

        def gather_b(j):
            return pltpu.make_async_copy(h_hbm.at[sidx.at[j]], rows_b, sem_b)

        gather_a(0).start()
        gather_b(1).start()
        npair = NCH // 2

        def body(i, c):
            gather_a(2 * i).wait()
            pltpu.sync_copy(rows_a, acc_sh.at[didx.at[2 * i]], add=True)

            @pl.when(i < npair - 1)
            def _():
                gather_a(2 * i + 2).start()

            gather_b(2 * i + 1).wait()
            pltpu.sync_copy(rows_b, acc_sh.at[didx.at[2 * i + 1]], add=True)

            @pl.when(i < npair - 1)
            def _():
                gather_b(2 * i + 3).start()

            return c

        lax.fori_loop(0, npair, body, 0)
        plsc.subcore_barrier()
        pltpu.sync_copy(acc_sh.at[pl.ds(sid * ZROWS, ZROWS)],
                        out_hbm.at[cid, pl.ds(sid * ZROWS, ZROWS)])

    return agg_kernel(h, src2d, dst2d)


_PREC = lax.Precision.DEFAULT
_RB = 2000


def _tc_dense1(x, W1, deg2d):
    K, F = x.shape[1], W1.shape[1]

    def body(x_ref, w_ref, deg_ref, ht_ref, dinv_ref):
        dinv = lax.rsqrt(deg_ref[...])
        h = jnp.dot(x_ref[...], w_ref[...],
                    preferred_element_type=jnp.float32, precision=_PREC)
        ht_ref[...] = h * dinv
        dinv_ref[...] = dinv

    return pl.pallas_call(
        body,
        grid=(N_NODES // _RB,),
        in_specs=[
            pl.BlockSpec((_RB, K), lambda i: (i, 0)),
            pl.BlockSpec((K, F), lambda i: (0, 0)),
            pl.BlockSpec((_RB, 1), lambda i: (i, 0)),
        ],
        out_specs=(
            pl.BlockSpec((_RB, F), lambda i: (i, 0)),
            pl.BlockSpec((_RB, 1), lambda i: (i, 0)),
        ),
        out_shape=(jax.ShapeDtypeStruct((N_NODES, F), jnp.float32),
                   jax.ShapeDtypeStruct((N_NODES, 1), jnp.float32)),
    )(x, W1, deg2d)


def _tc_dense2(S1p, ht1, dinv2d, b1, W2):
    K, F = W2.shape

    def body(sp_ref, ht_ref, dinv_ref, b_ref, w_ref, out_ref):
        s = sp_ref[0] + sp_ref[1] + ht_ref[...]
        dinv = dinv_ref[...]
        a = jnp.maximum(s * dinv + b_ref[...], 0.0)
        h2 = jnp.dot(a, w_ref[...],
                     preferred_element_type=jnp.float32, precision=_PREC)
        out_ref[...] = (h2 * dinv).astype(jnp.bfloat16)

    return pl.pallas_call(
        body,
        grid=(N_NODES // _RB,),
        in_specs=[
            pl.BlockSpec((2, _RB, K), lambda i: (0, i, 0)),
            pl.BlockSpec((_RB, K), lambda i: (i, 0)),
            pl.BlockSpec((_RB, 1), lambda i: (i, 0)),
            pl.BlockSpec((1, K), lambda i: (0, 0)),
            pl.BlockSpec((K, F), lambda i: (0, 0)),
        ],
        out_specs=pl.BlockSpec((_RB, F), lambda i: (i, 0)),
        out_shape=jax.ShapeDtypeStruct((N_NODES, F), jnp.bfloat16),
    )(S1p, ht1, dinv2d, b1, W2)


def _tc_final(S2p, ht2, dinv2d, b2, batch_row, Wl, bl):
    def body(sp_ref, ht_ref, dinv_ref, b_ref, batch_ref, wl_ref, bl_ref,
             out_ref):
        s = (sp_ref[0, :N_NODES, :].astype(jnp.float32)
             + sp_ref[1, :N_NODES, :].astype(jnp.float32)
             + ht_ref[...].astype(jnp.float32))
        h = jnp.maximum(s * dinv_ref[...] + b_ref[...], 0.0)
        gids = lax.broadcasted_iota(jnp.int32, (NUM_GRAPHS, N_NODES), 0)
        ohT = (gids == batch_ref[...]).astype(jnp.float32)
        sums = jnp.dot(ohT, h, preferred_element_type=jnp.float32,
                       precision=_PREC)
        counts = jnp.sum(ohT, axis=1, keepdims=True)
        g = sums / jnp.maximum(counts, 1.0)
        out_ref[...] = jnp.dot(g, wl_ref[...],
                               preferred_element_type=jnp.float32,
                               precision=_PREC) + bl_ref[...]

    return pl.pallas_call(
        body,
        out_shape=jax.ShapeDtypeStruct((NUM_GRAPHS, Wl.shape[1]), jnp.float32),
    )(S2p, ht2, dinv2d, b2, batch_row, Wl, bl)


def kernel(x, edge_index, batch, W1, b1, W2, b2, Wl, bl):
    src2d = edge_index[0].astype(jnp.int32).reshape(CROWS, CHUNK)
    dst2d = edge_index[1].astype(jnp.int32).reshape(CROWS, CHUNK)
    batch_row = batch.astype(jnp.int32).reshape(1, N_NODES)
    degp = _sc_degree(dst2d)
    deg2d = (degp[0, :N_NODES] + degp[1, :N_NODES] + 1.0)[:, None]
    ht1, dinv2d = _tc_dense1(x, W1, deg2d)
    S1p = _sc_aggregate(ht1, src2d, dst2d)
    ht2 = _tc_dense2(S1p, ht1, dinv2d, b1.reshape(1, -1), W2)
    S2p = _sc_aggregate(ht2, src2d, dst2d)
    return _tc_final(S2p, ht2, dinv2d, b2.reshape(1, -1), batch_row,
                     Wl, bl.reshape(1, -1))


# speedup vs baseline: 1.0618x; 1.0037x over previous
import functools

import jax
import jax.numpy as jnp
from jax import lax
from jax.experimental import pallas as pl
from jax.experimental.pallas import tpu as pltpu
from jax.experimental.pallas import tpu_sc as plsc

N_NODES = 10000
N_PAD = 10240
N_EDGES = 160000
NUM_GRAPHS = 64
NC, NS = 2, 16
NW = NC * NS
CHUNK = 125
NCH = 40
CROWS = N_EDGES // CHUNK
ZROWS = N_PAD // NS
ZCH = 128

_mesh = lambda: plsc.VectorSubcoreMesh(core_axis_name="c", subcore_axis_name="s")


def _sc_degree(dst2d):
    @functools.partial(
        pl.kernel,
        out_type=jax.ShapeDtypeStruct((NC, N_PAD), jnp.float32),
        mesh=_mesh(),
        compiler_params=pltpu.CompilerParams(use_tc_tiling_on_sc=False),
        scratch_types=(
            pltpu.VMEM((NCH, CHUNK), jnp.int32),
            pltpu.VMEM((128,), jnp.float32),
            pltpu.VMEM((ZROWS,), jnp.float32),
            pltpu.SemaphoreType.DMA,
            pltpu.VMEM_SHARED((N_PAD,), jnp.float32),
        ),
    )
    def deg_kernel(dst_hbm, out_hbm, didx, ones_v, zbuf, sem, deg_sh):
        cid = lax.axis_index("c")
        sid = lax.axis_index("s")
        wid = sid * NC + cid
        zeros16 = jnp.zeros((16,), jnp.float32)
        ones16 = jnp.ones((16,), jnp.float32)

        def zb(i, c):
            zbuf[pl.ds(i * 16, 16)] = zeros16
            return c

        lax.fori_loop(0, ZROWS // 16, zb, 0)

        def ob(i, c):
            ones_v[pl.ds(i * 16, 16)] = ones16
            return c

        lax.fori_loop(0, 128 // 16, ob, 0)
        pltpu.sync_copy(zbuf, deg_sh.at[pl.ds(sid * ZROWS, ZROWS)])
        plsc.subcore_barrier()
        pltpu.sync_copy(dst_hbm.at[pl.ds(wid * NCH, NCH)], didx)

        def ch(i, c):
            pltpu.sync_copy(ones_v.at[pl.ds(0, CHUNK)],
                            deg_sh.at[didx.at[i]], add=True)
            return c

        lax.fori_loop(0, NCH, ch, 0)
        plsc.subcore_barrier()
        pltpu.sync_copy(deg_sh.at[pl.ds(sid * ZROWS, ZROWS)],
                        out_hbm.at[cid, pl.ds(sid * ZROWS, ZROWS)])

    return deg_kernel(dst2d)


def _sc_aggregate(h, src2d, dst2d):
    D = h.shape[1]
    dt = h.dtype
    lanes = 32 if dt == jnp.bfloat16 else 16

    @functools.partial(
        pl.kernel,
        out_type=jax.ShapeDtypeStruct((NC, N_PAD, D), dt),
        mesh=_mesh(),
        compiler_params=pltpu.CompilerParams(use_tc_tiling_on_sc=False),
        scratch_types=(
            pltpu.VMEM((NCH, CHUNK), jnp.int32),
            pltpu.VMEM((NCH, CHUNK), jnp.int32),
            pltpu.VMEM((CHUNK, D), dt),
            pltpu.VMEM((CHUNK, D), dt),
            pltpu.SemaphoreType.DMA,
            pltpu.SemaphoreType.DMA,
            pltpu.VMEM_SHARED((N_PAD, D), dt),
        ),
    )
    def agg_kernel(h_hbm, src_hbm, dst_hbm, out_hbm,
                   sidx, didx, rows_a, rows_b, sem_a, sem_b, acc_sh):
        cid = lax.axis_index("c")
        sid = lax.axis_index("s")
        wid = sid * NC + cid
        zvec = jnp.zeros((lanes,), dt)

        def zb(i, c):
            for j in range(D // lanes):
                rows_a[i, pl.ds(j * lanes, lanes)] = zvec
            return c

        lax.fori_loop(0, CHUNK, zb, 0)
        for k in range(ZROWS // CHUNK):
            pltpu.sync_copy(
                rows_a, acc_sh.at[pl.ds(sid * ZROWS + k * CHUNK, CHUNK)])
        pltpu.sync_copy(
            rows_a.at[pl.ds(0, ZROWS - (ZROWS // CHUNK) * CHUNK)],
            acc_sh.at[pl.ds(sid * ZROWS + (ZROWS // CHUNK) * CHUNK,
                            ZROWS - (ZROWS // CHUNK) * CHUNK)])
        plsc.subcore_barrier()
        r0 = wid * NCH
        pltpu.sync_copy(src_hbm.at[pl.ds(r0, NCH)], sidx)
        pltpu.sync_copy(dst_hbm.at[pl.ds(r0, NCH)], didx)

        def gather_a(j):
            return pltpu.make_async_copy(h_hbm.at[sidx.at[j]], rows_a, sem_a)

        def gather_b(j):
            return pltpu.make_async_copy(h_hbm.at[sidx.at[j]], rows_b, sem_b)

        gather_a(0).start()
        gather_b(1).start()
        npair = NCH // 2

        def body(i, c):
            gather_a(2 * i).wait()
            pltpu.sync_copy(rows_a, acc_sh.at[didx.at[2 * i]], add=True)

            @pl.when(i < npair - 1)
            def _():
                gather_a(2 * i + 2).start()

            gather_b(2 * i + 1).wait()
            pltpu.sync_copy(rows_b, acc_sh.at[didx.at[2 * i + 1]], add=True)

            @pl.when(i < npair - 1)
            def _():
                gather_b(2 * i + 3).start()

            return c

        lax.fori_loop(0, npair, body, 0)
        plsc.subcore_barrier()
        pltpu.sync_copy(acc_sh.at[pl.ds(sid * ZROWS, ZROWS)],
                        out_hbm.at[cid, pl.ds(sid * ZROWS, ZROWS)])

    return agg_kernel(h, src2d, dst2d)


_PREC = lax.Precision.DEFAULT
_RB = 2000


def _tc_dense1(x, W1, dinv2d):
    K, F = x.shape[1], W1.shape[1]

    def body(x_ref, w_ref, dinv_ref, ht_ref):
        h = jnp.dot(x_ref[...], w_ref[...],
                    preferred_element_type=jnp.float32, precision=_PREC)
        ht_ref[...] = h * dinv_ref[...]

    return pl.pallas_call(
        body,
        grid=(N_NODES // _RB,),
        in_specs=[
            pl.BlockSpec((_RB, K), lambda i: (i, 0)),
            pl.BlockSpec((K, F), lambda i: (0, 0)),
            pl.BlockSpec((_RB, 1), lambda i: (i, 0)),
        ],
        out_specs=pl.BlockSpec((_RB, F), lambda i: (i, 0)),
        out_shape=jax.ShapeDtypeStruct((N_NODES, F), jnp.float32),
    )(x, W1, dinv2d)


def _tc_dense2(S1p, ht1, dinv2d, b1, W2):
    K, F = W2.shape

    def body(sp_ref, ht_ref, dinv_ref, b_ref, w_ref, out_ref):
        s = sp_ref[0] + sp_ref[1] + ht_ref[...]
        dinv = dinv_ref[...]
        a = jnp.maximum(s * dinv + b_ref[...], 0.0)
        h2 = jnp.dot(a, w_ref[...],
                     preferred_element_type=jnp.float32, precision=_PREC)
        out_ref[...] = (h2 * dinv).astype(jnp.bfloat16)

    return pl.pallas_call(
        body,
        grid=(N_NODES // _RB,),
        in_specs=[
            pl.BlockSpec((2, _RB, K), lambda i: (0, i, 0)),
            pl.BlockSpec((_RB, K), lambda i: (i, 0)),
            pl.BlockSpec((_RB, 1), lambda i: (i, 0)),
            pl.BlockSpec((1, K), lambda i: (0, 0)),
            pl.BlockSpec((K, F), lambda i: (0, 0)),
        ],
        out_specs=pl.BlockSpec((_RB, F), lambda i: (i, 0)),
        out_shape=jax.ShapeDtypeStruct((N_NODES, F), jnp.bfloat16),
    )(S1p, ht1, dinv2d, b1, W2)


def _tc_final(S2p, ht2, dinv2d, b2, batch_row, Wl, bl):
    def body(sp_ref, ht_ref, dinv_ref, b_ref, batch_ref, wl_ref, bl_ref,
             out_ref):
        s = (sp_ref[0, :N_NODES, :].astype(jnp.float32)
             + sp_ref[1, :N_NODES, :].astype(jnp.float32)
             + ht_ref[...].astype(jnp.float32))
        h = jnp.maximum(s * dinv_ref[...] + b_ref[...], 0.0)
        gids = lax.broadcasted_iota(jnp.int32, (NUM_GRAPHS, N_NODES), 0)
        ohT = (gids == batch_ref[...]).astype(jnp.float32)
        sums = jnp.dot(ohT, h, preferred_element_type=jnp.float32,
                       precision=_PREC)
        counts = jnp.sum(ohT, axis=1, keepdims=True)
        g = sums / jnp.maximum(counts, 1.0)
        out_ref[...] = jnp.dot(g, wl_ref[...],
                               preferred_element_type=jnp.float32,
                               precision=_PREC) + bl_ref[...]

    return pl.pallas_call(
        body,
        out_shape=jax.ShapeDtypeStruct((NUM_GRAPHS, Wl.shape[1]), jnp.float32),
    )(S2p, ht2, dinv2d, b2, batch_row, Wl, bl)


def kernel(x, edge_index, batch, W1, b1, W2, b2, Wl, bl):
    src2d = edge_index[0].astype(jnp.int32).reshape(CROWS, CHUNK)
    dst2d = edge_index[1].astype(jnp.int32).reshape(CROWS, CHUNK)
    batch_row = batch.astype(jnp.int32).reshape(1, N_NODES)
    degp = _sc_degree(dst2d)
    dinv2d = lax.rsqrt(degp[0, :N_NODES] + degp[1, :N_NODES] + 1.0)[:, None]
    ht1 = _tc_dense1(x, W1, dinv2d)
    S1p = _sc_aggregate(ht1, src2d, dst2d)
    ht2 = _tc_dense2(S1p, ht1, dinv2d, b1.reshape(1, -1), W2)
    S2p = _sc_aggregate(ht2, src2d, dst2d)
    return _tc_final(S2p, ht2, dinv2d, b2.reshape(1, -1), batch_row,
                     Wl, bl.reshape(1, -1))


# speedup vs baseline: 1.0634x; 1.0016x over previous
import functools

import jax
import jax.numpy as jnp
from jax import lax
from jax.experimental import pallas as pl
from jax.experimental.pallas import tpu as pltpu
from jax.experimental.pallas import tpu_sc as plsc

N_NODES = 10000
N_PAD = 10240
N_EDGES = 160000
NUM_GRAPHS = 64
NC, NS = 2, 16
NW = NC * NS
CHUNK = 125
NCH = 40
CROWS = N_EDGES // CHUNK
ZROWS = N_PAD // NS
ZCH = 128

_mesh = lambda: plsc.VectorSubcoreMesh(core_axis_name="c", subcore_axis_name="s")


def _sc_degree(dst2d):
    @functools.partial(
        pl.kernel,
        out_type=jax.ShapeDtypeStruct((NC, N_PAD), jnp.float32),
        mesh=_mesh(),
        compiler_params=pltpu.CompilerParams(use_tc_tiling_on_sc=False),
        scratch_types=(
            pltpu.VMEM((NCH, CHUNK), jnp.int32),
            pltpu.VMEM((128,), jnp.float32),
            pltpu.VMEM((ZROWS,), jnp.float32),
            pltpu.SemaphoreType.DMA,
            pltpu.VMEM_SHARED((N_PAD,), jnp.float32),
        ),
    )
    def deg_kernel(dst_hbm, out_hbm, didx, ones_v, zbuf, sem, deg_sh):
        cid = lax.axis_index("c")
        sid = lax.axis_index("s")
        wid = sid * NC + cid
        zeros16 = jnp.zeros((16,), jnp.float32)
        ones16 = jnp.ones((16,), jnp.float32)

        def zb(i, c):
            zbuf[pl.ds(i * 16, 16)] = zeros16
            return c

        lax.fori_loop(0, ZROWS // 16, zb, 0)

        def ob(i, c):
            ones_v[pl.ds(i * 16, 16)] = ones16
            return c

        lax.fori_loop(0, 128 // 16, ob, 0)
        pltpu.sync_copy(zbuf, deg_sh.at[pl.ds(sid * ZROWS, ZROWS)])
        plsc.subcore_barrier()
        pltpu.sync_copy(dst_hbm.at[pl.ds(wid * NCH, NCH)], didx)

        def ch(i, c):
            pltpu.sync_copy(ones_v.at[pl.ds(0, CHUNK)],
                            deg_sh.at[didx.at[i]], add=True)
            return c

        lax.fori_loop(0, NCH, ch, 0)
        plsc.subcore_barrier()
        pltpu.sync_copy(deg_sh.at[pl.ds(sid * ZROWS, ZROWS)],
                        out_hbm.at[cid, pl.ds(sid * ZROWS, ZROWS)])

    return deg_kernel(dst2d)


def _sc_aggregate(h, src2d, dst2d):
    D = h.shape[1]
    dt = h.dtype
    lanes = 32 if dt == jnp.bfloat16 else 16

    @functools.partial(
        pl.kernel,
        out_type=jax.ShapeDtypeStruct((NC, N_PAD, D), dt),
        mesh=_mesh(),
        compiler_params=pltpu.CompilerParams(use_tc_tiling_on_sc=False),
        scratch_types=(
            pltpu.VMEM((NCH, CHUNK), jnp.int32),
            pltpu.VMEM((NCH, CHUNK), jnp.int32),
            pltpu.VMEM((CHUNK, D), dt),
            pltpu.VMEM((CHUNK, D), dt),
            pltpu.SemaphoreType.DMA,
            pltpu.SemaphoreType.DMA,
            pltpu.VMEM_SHARED((N_PAD, D), dt),
        ),
    )
    def agg_kernel(h_hbm, src_hbm, dst_hbm, out_hbm,
                   sidx, didx, rows_a, rows_b, sem_a, sem_b, acc_sh):
        cid = lax.axis_index("c")
        sid = lax.axis_index("s")
        wid = sid * NC + cid
        zvec = jnp.zeros((lanes,), dt)

        def zb(i, c):
            for j in range(D // lanes):
                rows_a[i, pl.ds(j * lanes, lanes)] = zvec
            return c

        lax.fori_loop(0, CHUNK, zb, 0)
        for k in range(ZROWS // CHUNK):
            pltpu.sync_copy(
                rows_a, acc_sh.at[pl.ds(sid * ZROWS + k * CHUNK, CHUNK)])
        pltpu.sync_copy(
            rows_a.at[pl.ds(0, ZROWS - (ZROWS // CHUNK) * CHUNK)],
            acc_sh.at[pl.ds(sid * ZROWS + (ZROWS // CHUNK) * CHUNK,
                            ZROWS - (ZROWS // CHUNK) * CHUNK)])
        plsc.subcore_barrier()
        r0 = wid * NCH
        pltpu.sync_copy(src_hbm.at[pl.ds(r0, NCH)], sidx)
        pltpu.sync_copy(dst_hbm.at[pl.ds(r0, NCH)], didx)

        def gather_a(j):
            return pltpu.make_async_copy(h_hbm.at[sidx.at[j]], rows_a, sem_a)

        def gather_b(j):
            return pltpu.make_async_copy(h_hbm.at[sidx.at[j]], rows_b, sem_b)

        gather_a(0).start()
        gather_b(1).start()
        npair = NCH // 2

        def body(i, c):
            gather_a(2 * i).wait()
            pltpu.sync_copy(rows_a, acc_sh.at[didx.at[2 * i]], add=True)

            @pl.when(i < npair - 1)
            def _():
                gather_a(2 * i + 2).start()

            gather_b(2 * i + 1).wait()
            pltpu.sync_copy(rows_b, acc_sh.at[didx.at[2 * i + 1]], add=True)

            @pl.when(i < npair - 1)
            def _():
                gather_b(2 * i + 3).start()

            return c

        lax.fori_loop(0, npair, body, 0)
        plsc.subcore_barrier()
        pltpu.sync_copy(acc_sh.at[pl.ds(sid * ZROWS, ZROWS)],
                        out_hbm.at[cid, pl.ds(sid * ZROWS, ZROWS)])

    return agg_kernel(h, src2d, dst2d)


_PREC = lax.Precision.DEFAULT
_RB = 5000


def _tc_dense1(x, W1, dinv2d):
    K, F = x.shape[1], W1.shape[1]

    def body(x_ref, w_ref, dinv_ref, ht_ref):
        h = jnp.dot(x_ref[...], w_ref[...],
                    preferred_element_type=jnp.float32, precision=_PREC)
        ht_ref[...] = h * dinv_ref[...]

    return pl.pallas_call(
        body,
        grid=(N_NODES // _RB,),
        in_specs=[
            pl.BlockSpec((_RB, K), lambda i: (i, 0)),
            pl.BlockSpec((K, F), lambda i: (0, 0)),
            pl.BlockSpec((_RB, 1), lambda i: (i, 0)),
        ],
        out_specs=pl.BlockSpec((_RB, F), lambda i: (i, 0)),
        out_shape=jax.ShapeDtypeStruct((N_NODES, F), jnp.float32),
    )(x, W1, dinv2d)


def _tc_dense2(S1p, ht1, dinv2d, b1, W2):
    K, F = W2.shape

    def body(sp_ref, ht_ref, dinv_ref, b_ref, w_ref, out_ref):
        s = sp_ref[0] + sp_ref[1] + ht_ref[...]
        dinv = dinv_ref[...]
        a = jnp.maximum(s * dinv + b_ref[...], 0.0)
        h2 = jnp.dot(a, w_ref[...],
                     preferred_element_type=jnp.float32, precision=_PREC)
        out_ref[...] = (h2 * dinv).astype(jnp.bfloat16)

    return pl.pallas_call(
        body,
        grid=(N_NODES // _RB,),
        in_specs=[
            pl.BlockSpec((2, _RB, K), lambda i: (0, i, 0)),
            pl.BlockSpec((_RB, K), lambda i: (i, 0)),
            pl.BlockSpec((_RB, 1), lambda i: (i, 0)),
            pl.BlockSpec((1, K), lambda i: (0, 0)),
            pl.BlockSpec((K, F), lambda i: (0, 0)),
        ],
        out_specs=pl.BlockSpec((_RB, F), lambda i: (i, 0)),
        out_shape=jax.ShapeDtypeStruct((N_NODES, F), jnp.bfloat16),
    )(S1p, ht1, dinv2d, b1, W2)


def _tc_final(S2p, ht2, dinv2d, b2, batch_row, Wl, bl):
    def body(sp_ref, ht_ref, dinv_ref, b_ref, batch_ref, wl_ref, bl_ref,
             out_ref):
        s = (sp_ref[0, :N_NODES, :].astype(jnp.float32)
             + sp_ref[1, :N_NODES, :].astype(jnp.float32)
             + ht_ref[...].astype(jnp.float32))
        h = jnp.maximum(s * dinv_ref[...] + b_ref[...], 0.0)
        gids = lax.broadcasted_iota(jnp.int32, (NUM_GRAPHS, N_NODES), 0)
        ohT = (gids == batch_ref[...]).astype(jnp.float32)
        sums = jnp.dot(ohT, h, preferred_element_type=jnp.float32,
                       precision=_PREC)
        counts = jnp.sum(ohT, axis=1, keepdims=True)
        g = sums / jnp.maximum(counts, 1.0)
        out_ref[...] = jnp.dot(g, wl_ref[...],
                               preferred_element_type=jnp.float32,
                               precision=_PREC) + bl_ref[...]

    return pl.pallas_call(
        body,
        out_shape=jax.ShapeDtypeStruct((NUM_GRAPHS, Wl.shape[1]), jnp.float32),
    )(S2p, ht2, dinv2d, b2, batch_row, Wl, bl)


def kernel(x, edge_index, batch, W1, b1, W2, b2, Wl, bl):
    src2d = edge_index[0].astype(jnp.int32).reshape(CROWS, CHUNK)
    dst2d = edge_index[1].astype(jnp.int32).reshape(CROWS, CHUNK)
    batch_row = batch.astype(jnp.int32).reshape(1, N_NODES)
    degp = _sc_degree(dst2d)
    dinv2d = lax.rsqrt(degp[0, :N_NODES] + degp[1, :N_NODES] + 1.0)[:, None]
    ht1 = _tc_dense1(x, W1, dinv2d)
    S1p = _sc_aggregate(ht1, src2d, dst2d)
    ht2 = _tc_dense2(S1p, ht1, dinv2d, b1.reshape(1, -1), W2)
    S2p = _sc_aggregate(ht2, src2d, dst2d)
    return _tc_final(S2p, ht2, dinv2d, b2.reshape(1, -1), batch_row,
                     Wl, bl.reshape(1, -1))
